# baseline (speedup 1.0000x reference)
import functools

import jax
import jax.numpy as jnp
from jax import lax
from jax.experimental import pallas as pl
from jax.experimental.pallas import tpu as pltpu
from jax.experimental.pallas import tpu_sc as plsc

LANES = 16
NWORKERS = 32
GATHER_N = 128
CHUNK = 16
LG = 8


def _matmul_body(t_ref, w_ref, o_ref):
    res = jnp.dot(t_ref[...], w_ref[...], preferred_element_type=jnp.float32)
    o_ref[...] = res.reshape(res.shape[0] * 2, 128)


def _project_table(table, wmat):
    V, D = table.shape
    E = wmat.shape[1]
    assert E == 256
    blk = 10000
    assert V % blk == 0
    return pl.pallas_call(
        _matmul_body,
        grid=(V // blk,),
        in_specs=[
            pl.BlockSpec((blk, D), lambda i: (i, 0)),
            pl.BlockSpec((D, E), lambda i: (0, 0)),
        ],
        out_specs=pl.BlockSpec((2 * blk, 128), lambda i: (i, 0)),
        out_shape=jax.ShapeDtypeStruct((2 * V, 128), jnp.float32),
    )(table, wmat)


def _pad_body(t_ref, o_ref):
    blk = t_ref.shape[0]
    o_ref[...] = jnp.concatenate(
        [t_ref[...], jnp.zeros((blk, 28), jnp.int32)], axis=1)


def _pad_ids(ids):
    B, L = ids.shape
    blk = 2048
    return pl.pallas_call(
        _pad_body,
        grid=(B // blk,),
        in_specs=[pl.BlockSpec((blk, L), lambda i: (i, 0))],
        out_specs=pl.BlockSpec((blk, 128), lambda i: (i, 0)),
        out_shape=jax.ShapeDtypeStruct((B, 128), jnp.int32),
    )(ids)


def _make_sc_lookup(B, L, C, G):
    rows_per_w = B // NWORKERS
    assert rows_per_w % CHUNK == 0
    nchunk = rows_per_w // CHUNK
    ids_per_chunk = CHUNK * L
    ngather = ids_per_chunk // GATHER_N
    vecs_per_row = GATHER_N // LANES

    mesh = plsc.VectorSubcoreMesh(core_axis_name="c", subcore_axis_name="s",
                                  num_cores=2, num_subcores=16)

    @functools.partial(
        pl.kernel,
        out_type=jax.ShapeDtypeStruct((B, C), jnp.float32),
        mesh=mesh,
        compiler_params=pltpu.CompilerParams(use_tc_tiling_on_sc=False),
        scratch_types=[
            pltpu.VMEM((ids_per_chunk,), jnp.int32),
            pltpu.VMEM((ids_per_chunk,), jnp.int32),
            pltpu.VMEM((ids_per_chunk,), jnp.int32),
            pltpu.VMEM((ngather, GATHER_N), jnp.int32),
            pltpu.VMEM((ngather, GATHER_N), jnp.int32),
            pltpu.VMEM((ids_per_chunk, LANES), jnp.float32),
            pltpu.VMEM((ids_per_chunk, LANES), jnp.float32),
            pltpu.VMEM((L, LANES), jnp.float32),
            pltpu.VMEM((CHUNK, LANES), jnp.float32),
            pltpu.VMEM((2 * LANES,), jnp.float32),
            pltpu.VMEM((LANES,), jnp.float32),
            pltpu.SemaphoreType.DMA,
        ],
    )
    def sc_lookup(p3_hbm, ids_hbm, gpat_hbm, mtab_hbm, bias_hbm, out_hbm,
                  ids_v0, ids_v1, gpat_v, idx_v0, idx_v1, rows_v0, rows_v1,
                  mtab_v, out_v, fold_v, bias_v, sem):
        wid = lax.axis_index("s") * 2 + lax.axis_index("c")
        ids_bufs = (ids_v0, ids_v1)
        idx_bufs = (idx_v0, idx_v1)
        rows_bufs = (rows_v0, rows_v1)

        pltpu.sync_copy(gpat_hbm, gpat_v)
        pltpu.sync_copy(mtab_hbm, mtab_v)
        pltpu.sync_copy(bias_hbm, bias_v)
        fold_v[pl.ds(LANES, LANES)] = jnp.zeros((LANES,), jnp.float32)
        bias = bias_v[...]

        def stage_chunk(k):
            pb = k & 1
            ids_v, idx_v, rows_v = ids_bufs[pb], idx_bufs[pb], rows_bufs[pb]
            base_row = wid * rows_per_w + k * CHUNK
            pltpu.sync_copy(
                ids_hbm.at[pl.ds(pl.multiple_of(base_row * L, 8),
                                 ids_per_chunk)],
                ids_v)

            def build_row(j, _):
                def build_vec(u, _):
                    o = (j * vecs_per_row + u) * LANES
                    v = ids_v[pl.ds(o, LANES)]
                    gp = gpat_v[pl.ds(o, LANES)]
                    idx_v[j, pl.ds(u * LANES, LANES)] = v * G + gp
                    return 0
                lax.fori_loop(0, vecs_per_row, build_vec, 0)
                return 0
            lax.fori_loop(0, ngather, build_row, 0)

            def fire(j, _):
                pltpu.async_copy(
                    p3_hbm.at[idx_v.at[j]],
                    rows_v.at[pl.ds(pl.multiple_of(j * GATHER_N, 8),
                                    GATHER_N)],
                    sem)
                return 0
            lax.fori_loop(0, ngather, fire, 0)

        def consume_chunk(k):
            qb = k & 1
            rows_v = rows_bufs[qb]
            base_row = wid * rows_per_w + k * CHUNK
            pltpu.make_async_copy(
                p3_hbm.at[pl.ds(0, ids_per_chunk)], rows_v, sem).wait()

            def reduce_row(i, _):
                tok0 = i * L

                def acc_l(l4, acc):
                    base = l4 * 8
                    for dl in range(8):
                        acc = acc + (rows_v[tok0 + base + dl, :]
                                     * mtab_v[base + dl, :])
                    return acc

                s = lax.fori_loop(0, L // 8, acc_l,
                                  jnp.zeros((LANES,), jnp.float32))
                for sh in (2, 4, 8):
                    fold_v[pl.ds(0, LANES)] = s
                    s = s + fold_v[pl.ds(sh, LANES)]
                out_v[i, :] = s + bias
                return 0
            lax.fori_loop(0, CHUNK, reduce_row, 0)

            pltpu.sync_copy(
                out_v.at[:, pl.ds(0, C)],
                out_hbm.at[pl.ds(pl.multiple_of(base_row, 8), CHUNK)])

        stage_chunk(0)
        for k in range(1, nchunk):
            stage_chunk(k)
            consume_chunk(k - 1)
        consume_chunk(nchunk - 1)

    return sc_lookup


def kernel(input, table, W, b):
    B, L = input.shape
    V, D = table.shape
    C = W.shape[0]
    E = 256
    G = E // (LG * C)

    wmat = W.reshape(C, L, D).transpose(2, 1, 0).reshape(D, L * C)
    wmat = jnp.pad(wmat, ((0, 0), (0, E - L * C)))
    p = _project_table(table, wmat)
    p3 = p.reshape(V * G, LG * C)

    LP = 128
    ids_flat = _pad_ids(input).reshape(B * LP)
    ids_per_chunk = CHUNK * LP
    gpat = (jnp.arange(ids_per_chunk, dtype=jnp.int32) % LP) // LG
    lseq = jnp.arange(LP, dtype=jnp.int32)
    mtab = ((jnp.arange(LANES, dtype=jnp.int32)[None, :] // C
             == (lseq % LG)[:, None])
            & (lseq < L)[:, None]).astype(jnp.float32)
    bias16 = jnp.pad(b.astype(jnp.float32), (0, LANES - C))

    sc_lookup = _make_sc_lookup(B, LP, C, G)
    return sc_lookup(p3, ids_flat, gpat, mtab, bias16)


# speedup vs baseline: 8.1030x; 8.1030x over previous
import functools

import jax
import jax.numpy as jnp
from jax import lax
from jax.experimental import pallas as pl
from jax.experimental.pallas import tpu as pltpu
from jax.experimental.pallas import tpu_sc as plsc

LANES = 16
NWORKERS = 32
GATHER_N = 128
CHUNK = 32
LG = 8


def _matmul_body(t_ref, w_ref, o_ref):
    res = jnp.dot(t_ref[...], w_ref[...], preferred_element_type=jnp.float32)
    o_ref[...] = res.reshape(res.shape[0] * 2, 128)


def _project_table(table, wmat):
    V, D = table.shape
    E = wmat.shape[1]
    assert E == 256
    blk = 10000
    assert V % blk == 0
    return pl.pallas_call(
        _matmul_body,
        grid=(V // blk,),
        in_specs=[
            pl.BlockSpec((blk, D), lambda i: (i, 0)),
            pl.BlockSpec((D, E), lambda i: (0, 0)),
        ],
        out_specs=pl.BlockSpec((2 * blk, 128), lambda i: (i, 0)),
        out_shape=jax.ShapeDtypeStruct((2 * V, 128), jnp.float32),
    )(table, wmat)


def _make_sc_lookup(B, L, C, G):
    rows_per_w = B // NWORKERS
    assert rows_per_w % CHUNK == 0
    nchunk = rows_per_w // CHUNK
    ids_per_chunk = CHUNK * L
    ngather = ids_per_chunk // GATHER_N
    vecs_per_row = GATHER_N // LANES

    mesh = plsc.VectorSubcoreMesh(core_axis_name="c", subcore_axis_name="s",
                                  num_cores=2, num_subcores=16)

    @functools.partial(
        pl.kernel,
        out_type=jax.ShapeDtypeStruct((B, C), jnp.float32),
        mesh=mesh,
        compiler_params=pltpu.CompilerParams(use_tc_tiling_on_sc=False),
        scratch_types=[
            pltpu.VMEM((ids_per_chunk,), jnp.int32),
            pltpu.VMEM((ids_per_chunk,), jnp.int32),
            pltpu.VMEM((ids_per_chunk,), jnp.int32),
            pltpu.VMEM((ngather, GATHER_N), jnp.int32),
            pltpu.VMEM((ngather, GATHER_N), jnp.int32),
            pltpu.VMEM((ids_per_chunk, LANES), jnp.float32),
            pltpu.VMEM((ids_per_chunk, LANES), jnp.float32),
            pltpu.VMEM((L, LANES), jnp.float32),
            pltpu.VMEM((CHUNK, LANES), jnp.float32),
            pltpu.VMEM((2 * LANES,), jnp.float32),
            pltpu.VMEM((LANES,), jnp.float32),
            pltpu.SemaphoreType.DMA,
        ],
    )
    def sc_lookup(p3_hbm, ids_hbm, gpat_hbm, mtab_hbm, bias_hbm, out_hbm,
                  ids_v0, ids_v1, gpat_v, idx_v0, idx_v1, rows_v0, rows_v1,
                  mtab_v, out_v, fold_v, bias_v, sem):
        wid = lax.axis_index("s") * 2 + lax.axis_index("c")
        ids_bufs = (ids_v0, ids_v1)
        idx_bufs = (idx_v0, idx_v1)
        rows_bufs = (rows_v0, rows_v1)

        pltpu.sync_copy(gpat_hbm, gpat_v)
        pltpu.sync_copy(mtab_hbm, mtab_v)
        pltpu.sync_copy(bias_hbm, bias_v)
        fold_v[pl.ds(LANES, LANES)] = jnp.zeros((LANES,), jnp.float32)
        bias = bias_v[...]

        def stage_chunk(k):
            pb = k & 1
            ids_v, idx_v, rows_v = ids_bufs[pb], idx_bufs[pb], rows_bufs[pb]
            base_row = wid * rows_per_w + k * CHUNK
            pltpu.sync_copy(
                ids_hbm.at[pl.ds(pl.multiple_of(base_row * L, 8),
                                 ids_per_chunk)],
                ids_v)

            def build_row(j, _):
                def build_vec(u, _):
                    o = (j * vecs_per_row + u) * LANES
                    v = ids_v[pl.ds(o, LANES)]
                    gp = gpat_v[pl.ds(o, LANES)]
                    idx_v[j, pl.ds(u * LANES, LANES)] = v * G + gp
                    return 0
                lax.fori_loop(0, vecs_per_row, build_vec, 0)
                return 0
            lax.fori_loop(0, ngather, build_row, 0)

            def fire(j, _):
                pltpu.async_copy(
                    p3_hbm.at[idx_v.at[j]],
                    rows_v.at[pl.ds(pl.multiple_of(j * GATHER_N, 8),
                                    GATHER_N)],
                    sem)
                return 0
            lax.fori_loop(0, ngather, fire, 0)

        def consume_chunk(k):
            qb = k & 1
            rows_v = rows_bufs[qb]
            base_row = wid * rows_per_w + k * CHUNK
            pltpu.make_async_copy(
                p3_hbm.at[pl.ds(0, ids_per_chunk)], rows_v, sem).wait()

            def reduce_row(i, _):
                tok0 = i * L

                def acc_l(l4, acc):
                    base = l4 * 10
                    for dl in range(10):
                        acc = acc + (rows_v[tok0 + base + dl, :]
                                     * mtab_v[base + dl, :])
                    return acc

                s = lax.fori_loop(0, L // 10, acc_l,
                                  jnp.zeros((LANES,), jnp.float32))
                for sh in (2, 4, 8):
                    fold_v[pl.ds(0, LANES)] = s
                    s = s + fold_v[pl.ds(sh, LANES)]
                out_v[i, :] = s + bias
                return 0
            lax.fori_loop(0, CHUNK, reduce_row, 0)

            pltpu.sync_copy(
                out_v.at[:, pl.ds(0, C)],
                out_hbm.at[pl.ds(pl.multiple_of(base_row, 8), CHUNK)])

        stage_chunk(0)
        for k in range(1, nchunk):
            stage_chunk(k)
            consume_chunk(k - 1)
        consume_chunk(nchunk - 1)

    return sc_lookup


def kernel(input, table, W, b):
    B, L = input.shape
    V, D = table.shape
    C = W.shape[0]
    E = 256
    G = E // (LG * C)

    wmat = W.reshape(C, L, D).transpose(2, 1, 0).reshape(D, L * C)
    wmat = jnp.pad(wmat, ((0, 0), (0, E - L * C)))
    p = _project_table(table, wmat)
    p3 = p.reshape(V * G, LG * C)

    ids_flat = input.reshape(B * L)
    ids_per_chunk = CHUNK * L
    gpat = (jnp.arange(ids_per_chunk, dtype=jnp.int32) % L) // LG
    lpos = jnp.arange(L, dtype=jnp.int32) % LG
    mtab = (jnp.arange(LANES, dtype=jnp.int32)[None, :] // C
            == lpos[:, None]).astype(jnp.float32)
    bias16 = jnp.pad(b.astype(jnp.float32), (0, LANES - C))

    sc_lookup = _make_sc_lookup(B, L, C, G)
    return sc_lookup(p3, ids_flat, gpat, mtab, bias16)
